# baseline (speedup 1.0000x reference)
import jax
import jax.numpy as jnp
from jax import lax
from jax.experimental import pallas as pl
from jax.experimental.pallas import tpu as pltpu
from jax.experimental.pallas import tpu_sc as plsc

B = 16384
L = 50
LP = 64
D = 64
LANES = 16
V = 1000000
PACK = V // 2

NUM_CORES = 2
NUM_SUBCORES = 16
NW = NUM_CORES * NUM_SUBCORES

VBLK = 256
VPITCH = 260
N_FULL = V // VBLK
TAIL_V = V - N_FULL * VBLK

DOCS_PER_W = B // NW
CHUNK_DOCS = 8
CHUNK_ROWS = CHUNK_DOCS * L
GATHER_BLK = 80
N_BLKS = CHUNK_ROWS // GATHER_BLK
N_CHUNKS = DOCS_PER_W // CHUNK_DOCS
N_PAIRS = N_CHUNKS // 2


def _repack_kernel(tabt_hbm, tail_hbm, pack_hbm, in_v, out_v, sem_in,
                   sem_out):
    wid = lax.axis_index("s") * NUM_CORES + lax.axis_index("c")
    n_steps = N_FULL // NW + 1

    iotas = [(jnp.arange(LANES, dtype=jnp.int32) + k * LANES)
             for k in range(D // LANES)]

    def in_copies(q, b):
        return [
            pltpu.make_async_copy(
                tabt_hbm.at[pl.ds(8 * tr, 8), pl.ds(q * VBLK, VBLK)],
                in_v.at[b, pl.ds(8 * tr, 8), pl.ds(0, VBLK)], sem_in.at[b])
            for tr in range(D // 8)
        ]

    def out_copy(q, b):
        return pltpu.make_async_copy(
            out_v.at[b], pack_hbm.at[pl.ds(q * (VBLK // 2), VBLK // 2)],
            sem_out.at[b])

    def transpose_rows(b, nrows):
        ROWS = 4

        def jb(jj, _):
            vals = []
            for u in range(ROWS):
                j = jj * ROWS + u
                for h in range(2):
                    c = (j * 2 + h).astype(jnp.int32)
                    cidx = jnp.zeros((LANES,), jnp.int32) + c
                    for k in range(D // LANES):
                        vals.append(
                            plsc.load_gather(in_v.at[b], [iotas[k], cidx]))
            i = 0
            for u in range(ROWS):
                j = jj * ROWS + u
                for h in range(2):
                    for k in range(D // LANES):
                        out_v[b, j, pl.ds(h * D + k * LANES, LANES)] = vals[i]
                        i += 1
            return 0

        lax.fori_loop(0, nrows // ROWS, jb, 0)

    def step_body(i, _):
        for b in range(2):
            s = 2 * i + b
            q = s * NW + wid

            @pl.when(q < N_FULL)
            def _():
                for c in in_copies(q, b):
                    c.wait()

                @pl.when(s >= 2)
                def _():
                    out_copy(q - 2 * NW, b).wait()
                transpose_rows(b, VBLK // 2)
                @pl.when(q + 2 * NW < N_FULL)
                def _():
                    for c in in_copies(q + 2 * NW, b):
                        c.start()
                out_copy(q, b).start()
        return 0

    for b in range(2):
        q0 = b * NW + wid

        @pl.when(q0 < N_FULL)
        def _():
            for c in in_copies(q0, b):
                c.start()

    n_pairs = (n_steps + 1) // 2
    lax.fori_loop(0, n_pairs, step_body, 0)

    for t in range(n_steps - 3, n_steps):
        qt = t * NW + wid
        qt2 = (t + 2) * NW + wid

        @pl.when(jnp.logical_and(qt < N_FULL, qt2 >= N_FULL))
        def _():
            out_copy(qt, t % 2).wait()

    @pl.when(wid == 0)
    def _():
        pltpu.sync_copy(tail_hbm, out_v.at[0, pl.ds(0, TAIL_V // 2)])
        pltpu.sync_copy(out_v.at[0, pl.ds(0, TAIL_V // 2)],
                        pack_hbm.at[pl.ds(N_FULL * (VBLK // 2), TAIL_V // 2)])


def _gather_kernel(pack_hbm, idx_hbm, w_hbm, hof_hbm, out_hbm,
                   idx_v, w_v, hof_v, rows_v, out_v,
                   sem_in, sem_w, sem_h, sem_rows, sem_out):
    wid = lax.axis_index("s") * NUM_CORES + lax.axis_index("c")
    doc0 = wid * DOCS_PER_W

    def idx_copy(g, b):
        d0 = doc0 + g * CHUNK_DOCS
        return pltpu.make_async_copy(idx_hbm.at[pl.ds(d0 * L, CHUNK_ROWS)],
                                     idx_v.at[b], sem_in.at[b])

    def w_copy(g, b):
        d0 = doc0 + g * CHUNK_DOCS
        return pltpu.make_async_copy(w_hbm.at[pl.ds(d0, CHUNK_DOCS)],
                                     w_v.at[b], sem_w.at[b])

    def hof_copy(g, b):
        d0 = doc0 + g * CHUNK_DOCS
        return pltpu.make_async_copy(hof_hbm.at[pl.ds(d0, CHUNK_DOCS)],
                                     hof_v.at[b], sem_h.at[b])

    def gather_copies(b):
        return [
            pltpu.make_async_copy(
                pack_hbm.at[idx_v.at[b, pl.ds(j * GATHER_BLK, GATHER_BLK)]],
                rows_v.at[b, pl.ds(j * GATHER_BLK, GATHER_BLK)],
                sem_rows.at[b])
            for j in range(N_BLKS)
        ]

    def out_copy(g, b):
        d0 = doc0 + g * CHUNK_DOCS
        return pltpu.make_async_copy(out_v.at[b],
                                     out_hbm.at[pl.ds(d0, CHUNK_DOCS)],
                                     sem_out.at[b])

    for b in range(2):
        idx_copy(b, b).start()
        w_copy(b, b).start()
        hof_copy(b, b).start()
    for b in range(2):
        idx_copy(b, b).wait()
        for c in gather_copies(b):
            c.start()

    def compute(g, b):
        def doc_body(c, _):
            r0 = c * L
            wv = [w_v[b, c, pl.ds(i * LANES, LANES)]
                  for i in range(LP // LANES)]
            hv = [hof_v[b, c, pl.ds(i * LANES, LANES)]
                  for i in range(LP // LANES)]
            acc = [[jnp.zeros((LANES,), jnp.float32) for _ in range(2)]
                   for _ in range(D // LANES)]
            for l in range(L):
                w = wv[l // LANES][l % LANES]
                ho = hv[l // LANES][l % LANES]
                p = l % 2
                for k in range(D // LANES):
                    acc[k][p] = acc[k][p] + (
                        rows_v[b, r0 + l, pl.ds(ho + k * LANES, LANES)] * w)
            for k in range(D // LANES):
                out_v[b, c, pl.ds(k * LANES, LANES)] = acc[k][0] + acc[k][1]
            return 0

        lax.fori_loop(0, CHUNK_DOCS, doc_body, 0)

    def pair_body(i, _):
        for b in range(2):
            g = 2 * i + b
            for c in gather_copies(b):
                c.wait()
            @pl.when(i < N_PAIRS - 1)
            def _():
                idx_copy(g + 2, b).start()
            @pl.when(i > 0)
            def _():
                out_copy(g - 2, b).wait()
            w_copy(g, b).wait()
            hof_copy(g, b).wait()
            compute(g, b)
            out_copy(g, b).start()
            @pl.when(i < N_PAIRS - 1)
            def _():
                w_copy(g + 2, b).start()
                hof_copy(g + 2, b).start()
                idx_copy(g + 2, b).wait()
                for c in gather_copies(b):
                    c.start()
        return 0

    lax.fori_loop(0, N_PAIRS, pair_body, 0)
    for b in range(2):
        out_copy(N_CHUNKS - 2 + b, b).wait()


@jax.jit
def _run(table_t, tail_pack, idx_flat, w_pad, hof_pad):
    mesh = plsc.VectorSubcoreMesh(core_axis_name="c", subcore_axis_name="s")
    repack = pl.kernel(
        _repack_kernel,
        mesh=mesh,
        out_type=jax.ShapeDtypeStruct((PACK, 2 * D), jnp.float32),
        scratch_types=[
            pltpu.VMEM((2, D, VPITCH), jnp.float32),
            pltpu.VMEM((2, VBLK // 2, 2 * D), jnp.float32),
            pltpu.SemaphoreType.DMA((2,)),
            pltpu.SemaphoreType.DMA((2,)),
        ],
        compiler_params=pltpu.CompilerParams(use_tc_tiling_on_sc=True, needs_layout_passes=False),
    )
    pack = repack(table_t, tail_pack)

    gather = pl.kernel(
        _gather_kernel,
        mesh=mesh,
        out_type=jax.ShapeDtypeStruct((B, D), jnp.float32),
        scratch_types=[
            pltpu.VMEM((2, CHUNK_ROWS), jnp.int32),
            pltpu.VMEM((2, CHUNK_DOCS, LP), jnp.float32),
            pltpu.VMEM((2, CHUNK_DOCS, LP), jnp.int32),
            pltpu.VMEM((2, CHUNK_ROWS, 2 * D), jnp.float32),
            pltpu.VMEM((2, CHUNK_DOCS, D), jnp.float32),
            pltpu.SemaphoreType.DMA((2,)),
            pltpu.SemaphoreType.DMA((2,)),
            pltpu.SemaphoreType.DMA((2,)),
            pltpu.SemaphoreType.DMA((2,)),
            pltpu.SemaphoreType.DMA((2,)),
        ],
        compiler_params=pltpu.CompilerParams(use_tc_tiling_on_sc=False),
    )
    return gather(pack, idx_flat, w_pad, hof_pad)


def kernel(X, X_w, table):
    xi = X.astype(jnp.int32)
    idx_flat = (xi >> 1).reshape(-1)
    hof = (xi & 1) * D
    w_pad = jnp.pad(X_w.astype(jnp.float32), ((0, 0), (0, LP - L)))
    hof_pad = jnp.pad(hof, ((0, 0), (0, LP - L)))
    tail_pack = table[V - TAIL_V:, :].reshape(TAIL_V // 2, 2 * D)
    return _run(table.T, tail_pack, idx_flat, w_pad, hof_pad)


# speedup vs baseline: 1.6065x; 1.6065x over previous
import jax
import jax.numpy as jnp
from jax import lax
from jax.experimental import pallas as pl
from jax.experimental.pallas import tpu as pltpu
from jax.experimental.pallas import tpu_sc as plsc

B = 16384
L = 50
LP = 64
D = 64
LANES = 16
V = 1000000
PACK = V // 2

NUM_CORES = 2
NUM_SUBCORES = 16
NW = NUM_CORES * NUM_SUBCORES

VBLK = 256
VPITCH = 260
N_FULL = V // VBLK
TAIL_V = V - N_FULL * VBLK

DOCS_PER_W = B // NW
CHUNK_DOCS = 8
CHUNK_ROWS = CHUNK_DOCS * L
GATHER_BLK = 80
N_BLKS = CHUNK_ROWS // GATHER_BLK
N_CHUNKS = DOCS_PER_W // CHUNK_DOCS
N_PAIRS = N_CHUNKS // 2


def _gather_kernel(pack_hbm, idx_hbm, w_hbm, hof_hbm, out_hbm,
                   idx_v, w_v, hof_v, rows_v, out_v,
                   sem_in, sem_w, sem_h, sem_rows, sem_out):
    wid = lax.axis_index("s") * NUM_CORES + lax.axis_index("c")
    doc0 = wid * DOCS_PER_W

    def idx_copy(g, b):
        d0 = doc0 + g * CHUNK_DOCS
        return pltpu.make_async_copy(idx_hbm.at[pl.ds(d0 * L, CHUNK_ROWS)],
                                     idx_v.at[b], sem_in.at[b])

    def w_copy(g, b):
        d0 = doc0 + g * CHUNK_DOCS
        return pltpu.make_async_copy(w_hbm.at[pl.ds(d0, CHUNK_DOCS)],
                                     w_v.at[b], sem_w.at[b])

    def hof_copy(g, b):
        d0 = doc0 + g * CHUNK_DOCS
        return pltpu.make_async_copy(hof_hbm.at[pl.ds(d0, CHUNK_DOCS)],
                                     hof_v.at[b], sem_h.at[b])

    def gather_copies(b):
        return [
            pltpu.make_async_copy(
                pack_hbm.at[idx_v.at[b, pl.ds(j * GATHER_BLK, GATHER_BLK)]],
                rows_v.at[b, pl.ds(j * GATHER_BLK, GATHER_BLK)],
                sem_rows.at[b])
            for j in range(N_BLKS)
        ]

    def out_copy(g, b):
        d0 = doc0 + g * CHUNK_DOCS
        return pltpu.make_async_copy(out_v.at[b],
                                     out_hbm.at[pl.ds(d0, CHUNK_DOCS)],
                                     sem_out.at[b])

    for b in range(2):
        idx_copy(b, b).start()
        w_copy(b, b).start()
        hof_copy(b, b).start()
    for b in range(2):
        idx_copy(b, b).wait()
        for c in gather_copies(b):
            c.start()

    def compute(g, b):
        def doc_body(c, _):
            r0 = c * L
            wv = [w_v[b, c, pl.ds(i * LANES, LANES)]
                  for i in range(LP // LANES)]
            hv = [hof_v[b, c, pl.ds(i * LANES, LANES)]
                  for i in range(LP // LANES)]
            acc = [[jnp.zeros((LANES,), jnp.float32) for _ in range(2)]
                   for _ in range(D // LANES)]
            for l in range(L):
                w = wv[l // LANES][l % LANES]
                ho = hv[l // LANES][l % LANES]
                p = l % 2
                for k in range(D // LANES):
                    acc[k][p] = acc[k][p] + (
                        rows_v[b, r0 + l, pl.ds(ho + k * LANES, LANES)] * w)
            for k in range(D // LANES):
                out_v[b, c, pl.ds(k * LANES, LANES)] = acc[k][0] + acc[k][1]
            return 0

        lax.fori_loop(0, CHUNK_DOCS, doc_body, 0)

    def pair_body(i, _):
        for b in range(2):
            g = 2 * i + b
            for c in gather_copies(b):
                c.wait()
            @pl.when(i < N_PAIRS - 1)
            def _():
                idx_copy(g + 2, b).start()
            @pl.when(i > 0)
            def _():
                out_copy(g - 2, b).wait()
            w_copy(g, b).wait()
            hof_copy(g, b).wait()
            compute(g, b)
            out_copy(g, b).start()
            @pl.when(i < N_PAIRS - 1)
            def _():
                w_copy(g + 2, b).start()
                hof_copy(g + 2, b).start()
                idx_copy(g + 2, b).wait()
                for c in gather_copies(b):
                    c.start()
        return 0

    lax.fori_loop(0, N_PAIRS, pair_body, 0)
    for b in range(2):
        out_copy(N_CHUNKS - 2 + b, b).wait()


@jax.jit
def _run(pack, idx_flat, w_pad, hof_pad):
    mesh = plsc.VectorSubcoreMesh(core_axis_name="c", subcore_axis_name="s")
    gather = pl.kernel(
        _gather_kernel,
        mesh=mesh,
        out_type=jax.ShapeDtypeStruct((B, D), jnp.float32),
        scratch_types=[
            pltpu.VMEM((2, CHUNK_ROWS), jnp.int32),
            pltpu.VMEM((2, CHUNK_DOCS, LP), jnp.float32),
            pltpu.VMEM((2, CHUNK_DOCS, LP), jnp.int32),
            pltpu.VMEM((2, CHUNK_ROWS, 2 * D), jnp.float32),
            pltpu.VMEM((2, CHUNK_DOCS, D), jnp.float32),
            pltpu.SemaphoreType.DMA((2,)),
            pltpu.SemaphoreType.DMA((2,)),
            pltpu.SemaphoreType.DMA((2,)),
            pltpu.SemaphoreType.DMA((2,)),
            pltpu.SemaphoreType.DMA((2,)),
        ],
        compiler_params=pltpu.CompilerParams(use_tc_tiling_on_sc=False),
    )
    return gather(pack, idx_flat, w_pad, hof_pad)


def kernel(X, X_w, table):
    xi = X.astype(jnp.int32)
    idx_flat = (xi >> 1).reshape(-1)
    hof = (xi & 1) * D
    w_pad = jnp.pad(X_w.astype(jnp.float32), ((0, 0), (0, LP - L)))
    hof_pad = jnp.pad(hof, ((0, 0), (0, LP - L)))
    pack = table.reshape(PACK, 2 * D)
    return _run(pack, idx_flat, w_pad, hof_pad)
